# Initial kernel scaffold; baseline (speedup 1.0000x reference)
import jax
import jax.numpy as jnp
from jax.experimental import pallas as pl


def kernel(x1_features, x1_pos, x1_batch, x2_features, x2_pos, x2_batch, W1, b1, W2, b2):
    raise NotImplementedError("write your pallas kernel here")


# speedup vs baseline: 4.7801x; 4.7801x over previous
import functools

import jax
import jax.numpy as jnp
from jax import lax
from jax.experimental import pallas as pl
from jax.experimental.pallas import tpu as pltpu
from jax.experimental.pallas import tpu_sc as plsc

N1 = 10000
N2 = 10000
D = 128
K = 32
RSQ = 0.1 * 0.1
N1P = 10240
QB = 400
QD = 400
NEG = float(jnp.finfo(jnp.float32).min)


def _pq_body(x1f, x1p, x2f, x2p, w1, b1, p_out, q_out):
    wa = w1[0:D, :]
    wb = w1[D:2 * D, :]
    wc = w1[2 * D:2 * D + 3, :]
    f32 = jnp.float32
    p_out[...] = (jnp.dot(x1f[...], wb, preferred_element_type=f32)
                  + jnp.dot(x1p[...], wc, preferred_element_type=f32))
    q_out[...] = (jnp.dot(x2f[...], wa, preferred_element_type=f32)
                  - jnp.dot(x2p[...], wc, preferred_element_type=f32)
                  + b1[...][None, :])


def _compute_pq(x1f, x1p, x2f, x2p, w1, b1):
    return pl.pallas_call(
        _pq_body,
        out_shape=(jax.ShapeDtypeStruct((N1, D), jnp.float32),
                   jax.ShapeDtypeStruct((N2, D), jnp.float32)),
    )(x1f, x1p, x2f, x2p, w1, b1)


def _topk_body(x2p_ref, x1t_ref, idx_ref, val_ref, d2):
    qx = x2p_ref[:, 0:1]
    qy = x2p_ref[:, 1:2]
    qz = x2p_ref[:, 2:3]
    sx = x1t_ref[0:1, :]
    sy = x1t_ref[1:2, :]
    sz = x1t_ref[2:3, :]
    dx = qx - sx
    dy = qy - sy
    dz = qz - sz
    d2[...] = dx * dx + dy * dy + dz * dz
    lane = lax.broadcasted_iota(jnp.int32, (QB, N1P), 1)
    big = jnp.float32(3.0e38)
    for k in range(K):
        d = d2[...]
        m = jnp.min(d, axis=1, keepdims=True)
        cand = jnp.where(d == m, lane, jnp.int32(2**30))
        a = jnp.min(cand, axis=1, keepdims=True)
        idx_ref[:, k:k + 1] = a
        val_ref[:, k:k + 1] = (m <= jnp.float32(RSQ)).astype(jnp.int32)
        d2[...] = jnp.where(lane == a, big, d)


def _topk(x2_pos, x1t_pad):
    grid = N2 // QB
    return pl.pallas_call(
        _topk_body,
        grid=(grid,),
        in_specs=[
            pl.BlockSpec((QB, 3), lambda i: (i, 0)),
            pl.BlockSpec((3, N1P), lambda i: (0, 0)),
        ],
        out_specs=(pl.BlockSpec((QB, K), lambda i: (i, 0)),
                   pl.BlockSpec((QB, K), lambda i: (i, 0))),
        out_shape=(jax.ShapeDtypeStruct((N2, K), jnp.int32),
                   jax.ShapeDtypeStruct((N2, K), jnp.int32)),
        scratch_shapes=[pltpu.VMEM((QB, N1P), jnp.float32)],
    )(x2_pos, x1t_pad)


_SC_B = 327680
_SC_CH = 128
_SC_NCH = _SC_B // 32 // _SC_CH


def _make_sc_gather():
    mesh = plsc.VectorSubcoreMesh(core_axis_name="c", subcore_axis_name="s")
    info = plsc.get_sparse_core_info()
    nc = info.num_cores
    b_per_w = _SC_B // (nc * info.num_subcores)

    @functools.partial(
        pl.kernel,
        mesh=mesh,
        out_type=jax.ShapeDtypeStruct((_SC_B, D), jnp.float32),
        scratch_types=[
            pltpu.VMEM((_SC_CH,), jnp.int32),
            pltpu.VMEM((_SC_CH, D), jnp.float32),
            pltpu.SemaphoreType.DMA,
        ],
    )
    def gather(table_hbm, idx_hbm, out_hbm, idx_v, rows_v, sem):
        wid = lax.axis_index("s") * nc + lax.axis_index("c")
        base = wid * b_per_w

        def body(c, _):
            off = base + c * _SC_CH
            pltpu.sync_copy(idx_hbm.at[pl.ds(off, _SC_CH)], idx_v)
            pltpu.async_copy(table_hbm.at[idx_v], rows_v, sem).wait()
            pltpu.sync_copy(rows_v, out_hbm.at[pl.ds(off, _SC_CH)])
            return _

        lax.fori_loop(0, _SC_NCH, body, None)

    return gather


def _agg_body(e_ref, q_ref, vpen_ref, v_ref, w2_ref, b2_ref, out_ref):
    h1 = jnp.maximum(e_ref[...] + q_ref[...], 0.0)
    hf = h1.reshape(QD * K, D)
    h2 = jnp.dot(hf, w2_ref[...], preferred_element_type=jnp.float32)
    h2 = h2 + b2_ref[...][None, :] + vpen_ref[...]
    h3 = h2.reshape(QD, K, D)
    mx = jnp.max(h3, axis=1)
    has = jnp.max(v_ref[...], axis=1, keepdims=True) > 0
    out_ref[...] = jnp.where(has, mx, 0.0)


def _aggregate(e3, q3, vpen, valid, w2, b2):
    grid = N2 // QD
    return pl.pallas_call(
        _agg_body,
        grid=(grid,),
        in_specs=[
            pl.BlockSpec((QD, K, D), lambda i: (i, 0, 0)),
            pl.BlockSpec((QD, 1, D), lambda i: (i, 0, 0)),
            pl.BlockSpec((QD * K, 1), lambda i: (i, 0)),
            pl.BlockSpec((QD, K), lambda i: (i, 0)),
            pl.BlockSpec((D, D), lambda i: (0, 0)),
            pl.BlockSpec((D,), lambda i: (0,)),
        ],
        out_specs=pl.BlockSpec((QD, D), lambda i: (i, 0)),
        out_shape=jax.ShapeDtypeStruct((N2, D), jnp.float32),
    )(e3, q3, vpen, valid, w2, b2)


def kernel(x1_features, x1_pos, x1_batch, x2_features, x2_pos, x2_batch,
           W1, b1, W2, b2):
    p, q = _compute_pq(x1_features, x1_pos, x2_features, x2_pos, W1, b1)
    x1t_pad = jnp.pad(x1_pos.T, ((0, 0), (0, N1P - N1)),
                      constant_values=1.0e9)
    idx, valid = _topk(x2_pos, x1t_pad)
    idx_flat = jnp.pad(idx.reshape(-1), (0, _SC_B - N2 * K))
    e = _make_sc_gather()(p, idx_flat)
    e3 = e[:N2 * K].reshape(N2, K, D)
    q3 = q.reshape(N2, 1, D)
    vpen = ((valid == 0).astype(jnp.float32) * NEG).reshape(N2 * K, 1)
    out = _aggregate(e3, q3, vpen, valid, W2, b2)
    return (out, x2_pos, x2_batch)


# speedup vs baseline: 4.9128x; 1.0278x over previous
import functools

import jax
import jax.numpy as jnp
from jax import lax
from jax.experimental import pallas as pl
from jax.experimental.pallas import tpu as pltpu
from jax.experimental.pallas import tpu_sc as plsc

N1 = 10000
N2 = 10000
D = 128
K = 32
RSQ = 0.1 * 0.1
N1P = 10240
QB = 400
QD = 400
NEG = float(jnp.finfo(jnp.float32).min)


def _pq_body(x1f, x1p, x2f, x2p, w1, b1, p_out, q_out):
    wa = w1[0:D, :]
    wb = w1[D:2 * D, :]
    wc = w1[2 * D:2 * D + 3, :]
    f32 = jnp.float32
    p_out[...] = (jnp.dot(x1f[...], wb, preferred_element_type=f32)
                  + jnp.dot(x1p[...], wc, preferred_element_type=f32))
    q_out[...] = (jnp.dot(x2f[...], wa, preferred_element_type=f32)
                  - jnp.dot(x2p[...], wc, preferred_element_type=f32)
                  + b1[...][None, :])


def _compute_pq(x1f, x1p, x2f, x2p, w1, b1):
    return pl.pallas_call(
        _pq_body,
        out_shape=(jax.ShapeDtypeStruct((N1, D), jnp.float32),
                   jax.ShapeDtypeStruct((N2, D), jnp.float32)),
    )(x1f, x1p, x2f, x2p, w1, b1)


def _topk_body(x2p_ref, x1t_ref, idx_ref, val_ref):
    qx = x2p_ref[:, 0:1]
    qy = x2p_ref[:, 1:2]
    qz = x2p_ref[:, 2:3]
    sx = x1t_ref[0:1, :]
    sy = x1t_ref[1:2, :]
    sz = x1t_ref[2:3, :]
    dx = qx - sx
    dy = qy - sy
    dz = qz - sz
    d = dx * dx + dy * dy + dz * dz
    lane = lax.broadcasted_iota(jnp.int32, (QB, N1P), 1)
    big = jnp.float32(3.0e38)
    for k in range(K):
        m = jnp.min(d, axis=1, keepdims=True)
        cand = jnp.where(d == m, lane, jnp.int32(2**30))
        a = jnp.min(cand, axis=1, keepdims=True)
        idx_ref[:, k:k + 1] = a
        val_ref[:, k:k + 1] = (m <= jnp.float32(RSQ)).astype(jnp.int32)
        if k + 1 < K:
            d = jnp.where(lane == a, big, d)


def _topk(x2_pos, x1t_pad):
    grid = N2 // QB
    return pl.pallas_call(
        _topk_body,
        grid=(grid,),
        in_specs=[
            pl.BlockSpec((QB, 3), lambda i: (i, 0)),
            pl.BlockSpec((3, N1P), lambda i: (0, 0)),
        ],
        out_specs=(pl.BlockSpec((QB, K), lambda i: (i, 0)),
                   pl.BlockSpec((QB, K), lambda i: (i, 0))),
        out_shape=(jax.ShapeDtypeStruct((N2, K), jnp.int32),
                   jax.ShapeDtypeStruct((N2, K), jnp.int32)),
    )(x2_pos, x1t_pad)


_SC_B = 327680
_SC_CH = 128
_SC_NCH = _SC_B // 32 // _SC_CH


def _make_sc_gather():
    mesh = plsc.VectorSubcoreMesh(core_axis_name="c", subcore_axis_name="s")
    info = plsc.get_sparse_core_info()
    nc = info.num_cores
    b_per_w = _SC_B // (nc * info.num_subcores)

    @functools.partial(
        pl.kernel,
        mesh=mesh,
        out_type=jax.ShapeDtypeStruct((_SC_B, D), jnp.float32),
        scratch_types=[
            pltpu.VMEM((_SC_CH,), jnp.int32),
            pltpu.VMEM((_SC_CH,), jnp.int32),
            pltpu.VMEM((_SC_CH, D), jnp.float32),
            pltpu.VMEM((_SC_CH, D), jnp.float32),
            pltpu.SemaphoreType.DMA,
            pltpu.SemaphoreType.DMA,
        ],
    )
    def gather(table_hbm, idx_hbm, out_hbm,
               idx0, idx1, rows0, rows1, sem0, sem1):
        wid = lax.axis_index("s") * nc + lax.axis_index("c")
        base = wid * b_per_w

        def start(c, idx_v, rows_v, sem):
            pltpu.sync_copy(idx_hbm.at[pl.ds(base + c * _SC_CH, _SC_CH)],
                            idx_v)
            pltpu.async_copy(table_hbm.at[idx_v], rows_v, sem)

        def finish(c, idx_v, rows_v, sem):
            pltpu.make_async_copy(table_hbm.at[idx_v], rows_v, sem).wait()
            pltpu.sync_copy(rows_v, out_hbm.at[pl.ds(base + c * _SC_CH,
                                                     _SC_CH)])

        start(0, idx0, rows0, sem0)

        def body(i, _):
            c = 2 * i
            start(c + 1, idx1, rows1, sem1)
            finish(c, idx0, rows0, sem0)
            start(c + 2, idx0, rows0, sem0)
            finish(c + 1, idx1, rows1, sem1)
            return _

        lax.fori_loop(0, _SC_NCH // 2 - 1, body, None)
        c = _SC_NCH - 2
        start(c + 1, idx1, rows1, sem1)
        finish(c, idx0, rows0, sem0)
        finish(c + 1, idx1, rows1, sem1)

    return gather


def _agg_body(e_ref, q_ref, vpen_ref, v_ref, w2_ref, b2_ref, out_ref):
    h1 = jnp.maximum(e_ref[...] + q_ref[...], 0.0)
    hf = h1.reshape(QD * K, D)
    h2 = jnp.dot(hf, w2_ref[...], preferred_element_type=jnp.float32)
    h2 = h2 + b2_ref[...][None, :] + vpen_ref[...]
    h3 = h2.reshape(QD, K, D)
    mx = jnp.max(h3, axis=1)
    has = jnp.max(v_ref[...], axis=1, keepdims=True) > 0
    out_ref[...] = jnp.where(has, mx, 0.0)


def _aggregate(e3, q3, vpen, valid, w2, b2):
    grid = N2 // QD
    return pl.pallas_call(
        _agg_body,
        grid=(grid,),
        in_specs=[
            pl.BlockSpec((QD, K, D), lambda i: (i, 0, 0)),
            pl.BlockSpec((QD, 1, D), lambda i: (i, 0, 0)),
            pl.BlockSpec((QD * K, 1), lambda i: (i, 0)),
            pl.BlockSpec((QD, K), lambda i: (i, 0)),
            pl.BlockSpec((D, D), lambda i: (0, 0)),
            pl.BlockSpec((D,), lambda i: (0,)),
        ],
        out_specs=pl.BlockSpec((QD, D), lambda i: (i, 0)),
        out_shape=jax.ShapeDtypeStruct((N2, D), jnp.float32),
    )(e3, q3, vpen, valid, w2, b2)


def kernel(x1_features, x1_pos, x1_batch, x2_features, x2_pos, x2_batch,
           W1, b1, W2, b2):
    p, q = _compute_pq(x1_features, x1_pos, x2_features, x2_pos, W1, b1)
    x1t_pad = jnp.pad(x1_pos.T, ((0, 0), (0, N1P - N1)),
                      constant_values=1.0e9)
    idx, valid = _topk(x2_pos, x1t_pad)
    idx_flat = jnp.pad(idx.reshape(-1), (0, _SC_B - N2 * K))
    e = _make_sc_gather()(p, idx_flat)
    e3 = e[:N2 * K].reshape(N2, K, D)
    q3 = q.reshape(N2, 1, D)
    vpen = ((valid == 0).astype(jnp.float32) * NEG).reshape(N2 * K, 1)
    out = _aggregate(e3, q3, vpen, valid, W2, b2)
    return (out, x2_pos, x2_batch)
